# baseline (speedup 1.0000x reference)
import jax
import jax.numpy as jnp
from jax.experimental import pallas as pl


_BM = 1024


def _matmul_block(z_ref, w_ref, o_ref):
    o_ref[...] = jnp.dot(z_ref[...], w_ref[...],
                         preferred_element_type=jnp.float32)


def kernel(Z, W):
    M, K = Z.shape
    N = W.shape[1]
    return pl.pallas_call(
        _matmul_block,
        grid=(M // _BM,),
        in_specs=[
            pl.BlockSpec((_BM, K), lambda i: (i, 0)),
            pl.BlockSpec((K, N), lambda i: (0, 0)),
        ],
        out_specs=pl.BlockSpec((_BM, N), lambda i: (i, 0)),
        out_shape=jax.ShapeDtypeStruct((M, N), jnp.float32),
    )(Z, W)
